# baseline (speedup 1.0000x reference)
import functools

import jax
import jax.numpy as jnp
from jax import lax
from jax.experimental import pallas as pl
from jax.experimental.pallas import tpu as pltpu
from jax.experimental.pallas import tpu_sc as plsc

_B, _NQ, _NK, _DIM = 8, 32, 8192, 768
_ID = 64
_BK = 1024
_NKB = _NK // _BK
_SCALE = _ID ** (-0.5)
_NEG = -3.0e38

_NROW = _B * _NQ
_L = 16
_NW = 32
_RPW = _NROW // _NW


def _tc_scores_body(q_ref, wq_ref, k_ref, wk_ref, out_ref, qh_s):
    j = pl.program_id(1)

    @pl.when(j == 0)
    def _():
        qh_s[...] = lax.dot_general(
            q_ref[0], wq_ref[...], (((1,), (0,)), ((), ())),
            preferred_element_type=jnp.float32) * _SCALE

    kp = lax.dot_general(
        k_ref[0], wk_ref[...], (((1,), (0,)), ((), ())),
        preferred_element_type=jnp.float32)
    out_ref[0] = lax.dot_general(
        qh_s[...], kp, (((1,), (1,)), ((), ())),
        preferred_element_type=jnp.float32)


def _tc_scores(q, k, Wq, Wk):
    return pl.pallas_call(
        _tc_scores_body,
        grid=(_B, _NKB),
        in_specs=[
            pl.BlockSpec((1, _NQ, _DIM), lambda b, j: (b, 0, 0)),
            pl.BlockSpec((_DIM, _ID), lambda b, j: (0, 0)),
            pl.BlockSpec((1, _BK, _DIM), lambda b, j: (b, j, 0)),
            pl.BlockSpec((_DIM, _ID), lambda b, j: (0, 0)),
        ],
        out_specs=pl.BlockSpec((1, _NQ, _BK), lambda b, j: (b, 0, j)),
        out_shape=jax.ShapeDtypeStruct((_B, _NQ, _NK), jnp.float32),
        scratch_shapes=[pltpu.VMEM((_NQ, _ID), jnp.float32)],
    )(q, Wq, k, Wk)


def _bubble4(m, x):
    h1 = jnp.maximum(m[0], x)
    l1 = jnp.minimum(m[0], x)
    h2 = jnp.maximum(m[1], l1)
    l2 = jnp.minimum(m[1], l1)
    h3 = jnp.maximum(m[2], l2)
    l3 = jnp.minimum(m[2], l2)
    h4 = jnp.maximum(m[3], l3)
    return [h1, h2, h3, h4]


def _sc_body(scores_hbm, out_hbm, row_v):
    wid = lax.axis_index("s") * 2 + lax.axis_index("c")

    def row_loop(i, _carry):
        r = wid * _RPW + i
        base_w = r * _NK
        pltpu.sync_copy(scores_hbm.at[pl.ds(base_w, _NK)], row_v)

        neg = jnp.full((_L,), _NEG, jnp.float32)

        def p1(t, carry):
            ms = list(carry)
            base = t * (4 * _L)
            for qd in range(4):
                x = row_v[pl.ds(base + qd * _L, _L)]
                ms[qd * 4:qd * 4 + 4] = _bubble4(ms[qd * 4:qd * 4 + 4], x)
            return tuple(ms)

        ms = lax.fori_loop(0, _NK // (4 * _L), p1, (neg,) * 16)

        m = list(ms[0:4])
        for qd in range(1, 4):
            for v_ in ms[qd * 4:qd * 4 + 4]:
                m = _bubble4(m, v_)

        s1 = s2 = s3 = s4 = jnp.float32(_NEG)
        for t in range(4):
            for lane in range(_L):
                x = m[t][lane]
                h1 = jnp.maximum(s1, x)
                l1 = jnp.minimum(s1, x)
                h2 = jnp.maximum(s2, l1)
                l2 = jnp.minimum(s2, l1)
                h3 = jnp.maximum(s3, l2)
                l3 = jnp.minimum(s3, l2)
                s4 = jnp.maximum(s4, l3)
                s1, s2, s3 = h1, h2, h3
        g1, thr = s1, s4

        def p2(t, acc):
            x = row_v[pl.ds(t * _L, _L)]
            p = jnp.where(x >= thr, jnp.exp(x - g1), 0.0)
            row_v[pl.ds(t * _L, _L)] = p
            return acc + p

        acc = lax.fori_loop(0, _NK // _L, p2, jnp.zeros((_L,), jnp.float32))
        denom = jnp.float32(0.0)
        for lane in range(_L):
            denom = denom + acc[lane]
        invd = jnp.ones((_L,), jnp.float32) / jnp.broadcast_to(denom, (_L,))

        def p3(t, c):
            row_v[pl.ds(t * _L, _L)] = row_v[pl.ds(t * _L, _L)] * invd
            return c

        lax.fori_loop(0, _NK // _L, p3, 0)
        pltpu.sync_copy(row_v, out_hbm.at[pl.ds(base_w, _NK)])
        return 0

    lax.fori_loop(0, _RPW, row_loop, 0)


def _sc_topk_softmax(flat_scores):
    mesh = plsc.VectorSubcoreMesh(core_axis_name="c", subcore_axis_name="s")
    fn = functools.partial(
        pl.kernel,
        mesh=mesh,
        out_type=jax.ShapeDtypeStruct((_NROW * _NK,), jnp.float32),
        scratch_types=[pltpu.VMEM((_NK,), jnp.float32)],
    )(_sc_body)
    return fn(flat_scores)


@jax.jit
def _run(q, k, Wq, Wk):
    scores = _tc_scores(q, k, Wq, Wk)
    flat = scores.reshape(_NROW * _NK)
    out = _sc_topk_softmax(flat)
    return out.reshape(_B, 1, _NQ, _NK)


def kernel(q, k, v, Wq, Wk):
    del v
    return _run(q, k, Wq, Wk)


# speedup vs baseline: 1.3759x; 1.3759x over previous
import functools

import jax
import jax.numpy as jnp
from jax import lax
from jax.experimental import pallas as pl
from jax.experimental.pallas import tpu as pltpu
from jax.experimental.pallas import tpu_sc as plsc

_B, _NQ, _NK, _DIM = 8, 32, 8192, 768
_ID = 64
_BK = 1024
_NKB = _NK // _BK
_SCALE = _ID ** (-0.5)
_NEG = -3.0e38

_NROW = _B * _NQ
_L = 16
_NW = 32
_RPW = _NROW // _NW


def _tc_scores_body(q_ref, wq_ref, k_ref, wk_ref, out_ref, qh_s):
    j = pl.program_id(1)

    @pl.when(j == 0)
    def _():
        qh_s[...] = lax.dot_general(
            q_ref[0], wq_ref[...], (((1,), (0,)), ((), ())),
            preferred_element_type=jnp.float32) * _SCALE

    kp = lax.dot_general(
        k_ref[0], wk_ref[...], (((1,), (0,)), ((), ())),
        preferred_element_type=jnp.float32)
    out_ref[0] = lax.dot_general(
        qh_s[...], kp, (((1,), (1,)), ((), ())),
        preferred_element_type=jnp.float32)


def _tc_scores(q, k, Wq, Wk):
    return pl.pallas_call(
        _tc_scores_body,
        grid=(_B, _NKB),
        in_specs=[
            pl.BlockSpec((1, _NQ, _DIM), lambda b, j: (b, 0, 0)),
            pl.BlockSpec((_DIM, _ID), lambda b, j: (0, 0)),
            pl.BlockSpec((1, _BK, _DIM), lambda b, j: (b, j, 0)),
            pl.BlockSpec((_DIM, _ID), lambda b, j: (0, 0)),
        ],
        out_specs=pl.BlockSpec((1, _NQ, _BK), lambda b, j: (b, 0, j)),
        out_shape=jax.ShapeDtypeStruct((_B, _NQ, _NK), jnp.float32),
        scratch_shapes=[pltpu.VMEM((_NQ, _ID), jnp.float32)],
    )(q, Wq, k, Wk)


def _bubble4(m, x):
    h1 = jnp.maximum(m[0], x)
    l1 = jnp.minimum(m[0], x)
    h2 = jnp.maximum(m[1], l1)
    l2 = jnp.minimum(m[1], l1)
    h3 = jnp.maximum(m[2], l2)
    l3 = jnp.minimum(m[2], l2)
    h4 = jnp.maximum(m[3], l3)
    return [h1, h2, h3, h4]


def _sc_body(scores_hbm, out_hbm, row_v):
    wid = lax.axis_index("s") * 2 + lax.axis_index("c")

    def row_loop(i, _carry):
        r = wid * _RPW + i
        base_w = r * _NK
        pltpu.sync_copy(scores_hbm.at[pl.ds(base_w, _NK)], row_v)

        neg = jnp.full((_L,), _NEG, jnp.float32)

        def p1(t, carry):
            ms = list(carry)
            base = t * (4 * _L)
            for qd in range(4):
                x = row_v[pl.ds(base + qd * _L, _L)]
                ms[qd * 4:qd * 4 + 4] = _bubble4(ms[qd * 4:qd * 4 + 4], x)
            return tuple(ms)

        ms = lax.fori_loop(0, _NK // (4 * _L), p1, (neg,) * 16, unroll=2)

        m = list(ms[0:4])
        for qd in range(1, 4):
            for v_ in ms[qd * 4:qd * 4 + 4]:
                m = _bubble4(m, v_)

        s1 = s2 = s3 = s4 = jnp.float32(_NEG)
        for t in range(4):
            for lane in range(_L):
                x = m[t][lane]
                h1 = jnp.maximum(s1, x)
                l1 = jnp.minimum(s1, x)
                h2 = jnp.maximum(s2, l1)
                l2 = jnp.minimum(s2, l1)
                h3 = jnp.maximum(s3, l2)
                l3 = jnp.minimum(s3, l2)
                s4 = jnp.maximum(s4, l3)
                s1, s2, s3 = h1, h2, h3
        g1, thr = s1, s4

        def p2(t, acc):
            x = row_v[pl.ds(t * _L, _L)]
            p = jnp.where(x >= thr, jnp.exp(x - g1), 0.0)
            row_v[pl.ds(t * _L, _L)] = p
            return acc + p

        acc = lax.fori_loop(0, _NK // _L, p2, jnp.zeros((_L,), jnp.float32),
                            unroll=8)
        denom = jnp.float32(0.0)
        for lane in range(_L):
            denom = denom + acc[lane]
        invd = jnp.ones((_L,), jnp.float32) / jnp.broadcast_to(denom, (_L,))

        def p3(t, c):
            row_v[pl.ds(t * _L, _L)] = row_v[pl.ds(t * _L, _L)] * invd
            return c

        lax.fori_loop(0, _NK // _L, p3, 0, unroll=8)
        pltpu.sync_copy(row_v, out_hbm.at[pl.ds(base_w, _NK)])
        return 0

    lax.fori_loop(0, _RPW, row_loop, 0)


def _sc_topk_softmax(flat_scores):
    mesh = plsc.VectorSubcoreMesh(core_axis_name="c", subcore_axis_name="s")
    fn = functools.partial(
        pl.kernel,
        mesh=mesh,
        out_type=jax.ShapeDtypeStruct((_NROW * _NK,), jnp.float32),
        scratch_types=[pltpu.VMEM((_NK,), jnp.float32)],
    )(_sc_body)
    return fn(flat_scores)


@jax.jit
def _run(q, k, Wq, Wk):
    scores = _tc_scores(q, k, Wq, Wk)
    flat = scores.reshape(_NROW * _NK)
    out = _sc_topk_softmax(flat)
    return out.reshape(_B, 1, _NQ, _NK)


def kernel(q, k, v, Wq, Wk):
    del v
    return _run(q, k, Wq, Wk)


# speedup vs baseline: 1.5939x; 1.1584x over previous
import functools

import jax
import jax.numpy as jnp
from jax import lax
from jax.experimental import pallas as pl
from jax.experimental.pallas import tpu as pltpu
from jax.experimental.pallas import tpu_sc as plsc

_B, _NQ, _NK, _DIM = 8, 32, 8192, 768
_ID = 64
_BK = 2048
_NKB = _NK // _BK
_SCALE = _ID ** (-0.5)
_NEG = -3.0e38

_NROW = _B * _NQ
_L = 16
_NW = 32
_RPW = _NROW // _NW


def _tc_scores_body(q_ref, wq_ref, k_ref, wk_ref, out_ref, qh_s):
    j = pl.program_id(1)

    @pl.when(j == 0)
    def _():
        qh_s[...] = lax.dot_general(
            q_ref[0], wq_ref[...], (((1,), (0,)), ((), ())),
            preferred_element_type=jnp.float32) * _SCALE

    kp = lax.dot_general(
        k_ref[0], wk_ref[...], (((1,), (0,)), ((), ())),
        preferred_element_type=jnp.float32)
    out_ref[0] = lax.dot_general(
        qh_s[...], kp, (((1,), (1,)), ((), ())),
        preferred_element_type=jnp.float32)


def _tc_scores(q, k, Wq, Wk):
    return pl.pallas_call(
        _tc_scores_body,
        grid=(_B, _NKB),
        in_specs=[
            pl.BlockSpec((1, _NQ, _DIM), lambda b, j: (b, 0, 0)),
            pl.BlockSpec((_DIM, _ID), lambda b, j: (0, 0)),
            pl.BlockSpec((1, _BK, _DIM), lambda b, j: (b, j, 0)),
            pl.BlockSpec((_DIM, _ID), lambda b, j: (0, 0)),
        ],
        out_specs=pl.BlockSpec((1, _NQ, _BK), lambda b, j: (b, 0, j)),
        out_shape=jax.ShapeDtypeStruct((_B, _NQ, _NK), jnp.float32),
        scratch_shapes=[pltpu.VMEM((_NQ, _ID), jnp.float32)],
    )(q, Wq, k, Wk)


def _bubble4(m, x):
    h1 = jnp.maximum(m[0], x)
    l1 = jnp.minimum(m[0], x)
    h2 = jnp.maximum(m[1], l1)
    l2 = jnp.minimum(m[1], l1)
    h3 = jnp.maximum(m[2], l2)
    l3 = jnp.minimum(m[2], l2)
    h4 = jnp.maximum(m[3], l3)
    return [h1, h2, h3, h4]


def _sc_body(scores_hbm, out_hbm, row_v):
    wid = lax.axis_index("s") * 2 + lax.axis_index("c")

    def row_loop(i, _carry):
        r = wid * _RPW + i
        base_w = r * _NK
        pltpu.sync_copy(scores_hbm.at[pl.ds(base_w, _NK)], row_v)

        neg = jnp.full((_L,), _NEG, jnp.float32)

        def p1(t, carry):
            ms = list(carry)
            base = t * (4 * _L)
            for qd in range(4):
                x = row_v[pl.ds(base + qd * _L, _L)]
                ms[qd * 4:qd * 4 + 4] = _bubble4(ms[qd * 4:qd * 4 + 4], x)
            return tuple(ms)

        ms = lax.fori_loop(0, _NK // (4 * _L), p1, (neg,) * 16, unroll=2)

        m = list(ms[0:4])
        for qd in range(1, 4):
            for v_ in ms[qd * 4:qd * 4 + 4]:
                m = _bubble4(m, v_)

        s1 = s2 = s3 = s4 = jnp.float32(_NEG)
        for t in range(4):
            for lane in range(_L):
                x = m[t][lane]
                h1 = jnp.maximum(s1, x)
                l1 = jnp.minimum(s1, x)
                h2 = jnp.maximum(s2, l1)
                l2 = jnp.minimum(s2, l1)
                h3 = jnp.maximum(s3, l2)
                l3 = jnp.minimum(s3, l2)
                s4 = jnp.maximum(s4, l3)
                s1, s2, s3 = h1, h2, h3
        g1, thr = s1, s4

        def p2(t, acc):
            x = row_v[pl.ds(t * _L, _L)]
            p = jnp.where(x >= thr, jnp.exp(x - g1), 0.0)
            row_v[pl.ds(t * _L, _L)] = p
            return acc + p

        acc = lax.fori_loop(0, _NK // _L, p2, jnp.zeros((_L,), jnp.float32),
                            unroll=16)
        denom = jnp.float32(0.0)
        for lane in range(_L):
            denom = denom + acc[lane]
        invd = jnp.ones((_L,), jnp.float32) / jnp.broadcast_to(denom, (_L,))

        def p3(t, c):
            row_v[pl.ds(t * _L, _L)] = row_v[pl.ds(t * _L, _L)] * invd
            return c

        lax.fori_loop(0, _NK // _L, p3, 0, unroll=16)
        pltpu.sync_copy(row_v, out_hbm.at[pl.ds(base_w, _NK)])
        return 0

    lax.fori_loop(0, _RPW, row_loop, 0)


def _sc_topk_softmax(flat_scores):
    mesh = plsc.VectorSubcoreMesh(core_axis_name="c", subcore_axis_name="s")
    fn = functools.partial(
        pl.kernel,
        mesh=mesh,
        out_type=jax.ShapeDtypeStruct((_NROW * _NK,), jnp.float32),
        scratch_types=[pltpu.VMEM((_NK,), jnp.float32)],
    )(_sc_body)
    return fn(flat_scores)


@jax.jit
def _run(q, k, Wq, Wk):
    scores = _tc_scores(q, k, Wq, Wk)
    flat = scores.reshape(_NROW * _NK)
    out = _sc_topk_softmax(flat)
    return out.reshape(_B, 1, _NQ, _NK)


def kernel(q, k, v, Wq, Wk):
    del v
    return _run(q, k, Wq, Wk)
